# baseline (speedup 1.0000x reference)
import jax
import jax.numpy as jnp
from jax import lax
from jax.experimental import pallas as pl

F32 = jnp.float32
HP = lax.Precision.HIGHEST
LP = lax.Precision.DEFAULT
N_IMG = 4
NF = 96
LD = 64
NE = 64
FLAT = 56 * 56 * 96


def _dot(a, b, prec=HP):
    return lax.dot_general(a, b, (((1,), (0,)), ((), ())), precision=prec,
                           preferred_element_type=F32)


def _l0_kernel(p_ref, w_ref, aff_ref, o_ref):
    ops = []
    for du in (0, 1):
        for dv in (0, 1):
            ops.append(p_ref[0, pl.ds(du, 56), pl.ds(dv, 56), :])
    g = jnp.concatenate(ops, axis=-1).reshape(56 * 56, 192)
    m = None
    for ab in range(4):
        y = _dot(g, w_ref[ab], LP)
        m = y if m is None else jnp.maximum(m, y)
    bias = aff_ref[0:1, :]
    scale = aff_ref[1:2, :]
    shift = aff_ref[2:3, :]
    res = jnp.maximum(m + bias, 0.0) * scale + shift
    o_ref[0, :56, :56, :] = res.reshape(56, 56, 512)
    o_ref[0, 56:57, :, :] = jnp.zeros((1, 57, 512), F32)
    o_ref[0, :, 56:57, :] = jnp.zeros((57, 1, 512), F32)


def _l1_kernel(p_ref, w_ref, aff_ref, o_ref):
    quads = []
    for a in (0, 1):
        for b in (0, 1):
            acc = None
            for di in (0, 1):
                for dj in (0, 1):
                    si, tj = a + di, b + dj
                    lane = ((si % 2) * 2 + (tj % 2)) * 128
                    op = p_ref[0, pl.ds(si // 2, 56),
                               pl.ds(tj // 2, 56), pl.ds(lane, 128)]
                    part = _dot(op.reshape(56 * 56, 128), w_ref[di * 2 + dj], LP)
                    acc = part if acc is None else acc + part
            quads.append(acc)
    m = jnp.maximum(jnp.maximum(quads[0], quads[1]),
                    jnp.maximum(quads[2], quads[3]))
    bias = aff_ref[0:1, :]
    scale = aff_ref[1:2, :]
    shift = aff_ref[2:3, :]
    res = jnp.maximum(m + bias, 0.0) * scale + shift
    o_ref[0, :56, :56, :] = res.reshape(56, 56, 96)
    o_ref[0, 56:57, :, :] = jnp.zeros((1, 57, 96), F32)
    o_ref[0, :, 56:57, :] = jnp.zeros((57, 1, 96), F32)


def _conv4_body(p_ref, w_ref, aff_ref, prec):
    acc = None
    for di in (0, 1):
        for dj in (0, 1):
            op = p_ref[0, pl.ds(di, 56), pl.ds(dj, 56), :]
            part = _dot(op.reshape(56 * 56, 96), w_ref[di * 2 + dj], prec)
            acc = part if acc is None else acc + part
    bias = aff_ref[0:1, :]
    scale = aff_ref[1:2, :]
    shift = aff_ref[2:3, :]
    return (jnp.maximum(acc + bias, 0.0) * scale + shift).reshape(56, 56, 96)


def _conv4_flat_kernel(p_ref, w_ref, aff_ref, o_ref):
    res = _conv4_body(p_ref, w_ref, aff_ref, LP)
    o_ref[0] = jnp.concatenate([res[:, j, :] for j in range(56)], axis=-1)


def _conv4_padlow_kernel(p_ref, w_ref, aff_ref, o_ref):
    x = p_ref[0]
    xp = jnp.concatenate([jnp.zeros((1, 56, 96), F32), x], axis=0)
    xp = jnp.concatenate([jnp.zeros((57, 1, 96), F32), xp], axis=1)
    acc = None
    for di in (0, 1):
        for dj in (0, 1):
            op = xp[di:di + 56, dj:dj + 56, :]
            part = _dot(op.reshape(56 * 56, 96), w_ref[di * 2 + dj], LP)
            acc = part if acc is None else acc + part
    bias = aff_ref[0:1, :]
    scale = aff_ref[1:2, :]
    shift = aff_ref[2:3, :]
    res = (jnp.maximum(acc + bias, 0.0) * scale + shift).reshape(56, 56, 96)
    o_ref[0, 1:57, 1:57, :] = res
    o_ref[0, 0:1, :, :] = jnp.zeros((1, 57, 96), F32)
    o_ref[0, :, 0:1, :] = jnp.zeros((57, 1, 96), F32)


def _dense_vq_kernel(h_ref, w_ref, db_ref, emb_ref, o_ref):
    k = pl.program_id(0)
    part = None
    for r in range(8):
        t = _dot(h_ref[:, r, :], w_ref[r], LP)
        part = t if part is None else part + t

    @pl.when(k == 0)
    def _():
        o_ref[...] = part

    @pl.when(k > 0)
    def _():
        o_ref[...] = o_ref[...] + part

    @pl.when(k == pl.num_programs(0) - 1)
    def _():
        z = o_ref[...] + db_ref[...]
        emb = emb_ref[...]
        e2 = jnp.sum(emb * emb, axis=0, keepdims=True)
        d = e2 - 2.0 * _dot(z, emb)
        mn = jnp.min(d, axis=1, keepdims=True)
        iota = lax.broadcasted_iota(jnp.int32, d.shape, 1)
        idx = jnp.min(jnp.where(d <= mn, iota, NE), axis=1, keepdims=True)
        onehot = (iota == idx).astype(F32)
        zq = lax.dot_general(onehot, emb, (((1,), (1,)), ((), ())),
                             precision=HP, preferred_element_type=F32)
        o_ref[...] = zq


def _dense_dec_kernel(z_ref, w_ref, db_ref, o_ref):
    y = _dot(z_ref[...], w_ref[...], LP) + db_ref[0]
    o_ref[...] = jnp.maximum(y, 0.0)


_DL1_TERMS = {
    (0, 0): [(0, 0, 0), (0, 1, 1), (1, 0, 2), (1, 1, 3)],
    (0, 1): [(0, 1, 4), (1, 1, 5)],
    (1, 0): [(1, 0, 6), (1, 1, 7)],
    (1, 1): [(1, 1, 8)],
}


def _dl1_kernel(p_ref, w_ref, aff_ref, o_ref):
    bias = aff_ref[0:1, :]
    scale = aff_ref[1:2, :]
    shift = aff_ref[2:3, :]
    for a in (0, 1):
        for b in (0, 1):
            acc = None
            for (ro, co, wi) in _DL1_TERMS[(a, b)]:
                op = p_ref[0, pl.ds(ro, 56), pl.ds(co, 56), :]
                part = _dot(op.reshape(56 * 56, 96), w_ref[wi], LP)
                acc = part if acc is None else acc + part
            res = jnp.maximum(acc + bias, 0.0) * scale + shift
            qi = a * 2 + b
            o_ref[0, qi, 1:57, 1:57, :] = res.reshape(56, 56, 96)
            o_ref[0, qi, 0:1, :, :] = jnp.zeros((1, 57, 96), F32)
            o_ref[0, qi, :, 0:1, :] = jnp.zeros((57, 1, 96), F32)


_ROWVARS = [(1, 0), (0, 1), (1, 1)]


def _final_kernel(p_ref, w_ref, fb_ref, o_ref):
    s = pl.program_id(1)
    rs = s * 28
    acc = None
    for rv, (a, ro) in enumerate(_ROWVARS):
        for cv, (b, co) in enumerate(_ROWVARS):
            op = p_ref[0, a * 2 + b, pl.ds(rs + ro, 28), pl.ds(co, 56), :]
            part = _dot(op.reshape(28 * 56, 96), w_ref[rv * 3 + cv], LP)
            acc = part if acc is None else acc + part
    y = acc + fb_ref[...]
    o_ref[0] = (1.0 / (1.0 + jnp.exp(-y))).reshape(28, 56, 48)


def _s2d(x):
    n, h, w, c = x.shape
    return (x.reshape(n, h // 2, 2, w // 2, 2, c)
             .transpose(0, 1, 3, 2, 4, 5)
             .reshape(n, h // 2, w // 2, 4 * c))


def _affine(g, b, m, v, pad=None):
    scale = g / jnp.sqrt(v + 1e-3)
    shift = b - m * scale
    if pad:
        z = jnp.zeros((pad - g.shape[0],), F32)
        return lambda bias: jnp.stack([jnp.concatenate([bias, z]),
                                       jnp.concatenate([scale, z]),
                                       jnp.concatenate([shift, z])])
    return lambda bias: jnp.stack([bias, scale, shift])


def _conv_stage(kfn, inp, w, aff, out_shape, grid, in_spec_shape, out_spec):
    ng = len(grid)

    def in_map(n, *r):
        return (n,) + (0,) * (len(in_spec_shape) - 1)

    return pl.pallas_call(
        kfn,
        grid=grid,
        in_specs=[
            pl.BlockSpec(in_spec_shape, in_map),
            pl.BlockSpec(w.shape, lambda *a: (0,) * w.ndim),
            pl.BlockSpec((3, aff.shape[1]), lambda *a: (0, 0)),
        ],
        out_specs=out_spec,
        out_shape=jax.ShapeDtypeStruct(out_shape, F32),
    )(inp, w, aff)


def kernel(x, params):
    p = params
    k0 = p['enc_k0']
    ii = {'ab': [], 'du': [], 'dv': [], 'rm': [], 'tm': [], 'po': [],
          'di': [], 'dj': []}
    for a in (0, 1):
        for b in (0, 1):
            for po_i in (0, 1):
                for po_j in (0, 1):
                    for di in (0, 1):
                        for dj in (0, 1):
                            s = 2 * po_i + a + di
                            t = 2 * po_j + b + dj
                            ii['ab'].append(a * 2 + b)
                            ii['du'].append(s // 4)
                            ii['dv'].append(t // 4)
                            ii['rm'].append(s % 4)
                            ii['tm'].append(t % 4)
                            ii['po'].append(po_i * 2 + po_j)
                            ii['di'].append(di)
                            ii['dj'].append(dj)
    iarr = {kk: jnp.array(vv, jnp.int32) for kk, vv in ii.items()}
    z0 = jnp.zeros((4, 2, 2, 4, 4, 3, 4, 128), F32)
    z0 = z0.at[iarr['ab'], iarr['du'], iarr['dv'], iarr['rm'], iarr['tm'],
               :, iarr['po'], :96].add(k0[iarr['di'], iarr['dj']])
    w0 = z0.reshape(4, 192, 512)
    aff0 = _affine(p['enc_bn_g0'], p['enc_bn_b0'], p['enc_bn_m0'],
                   p['enc_bn_v0'], pad=128)(p['enc_b0'])
    aff0 = jnp.tile(aff0, (1, 4))

    k1 = p['enc_k1']
    w1 = jnp.zeros((4, 128, 96), F32)
    for di in (0, 1):
        for dj in (0, 1):
            w1 = w1.at[di * 2 + dj, :96, :].set(k1[di, dj])
    aff1 = _affine(p['enc_bn_g1'], p['enc_bn_b1'], p['enc_bn_m1'],
                   p['enc_bn_v1'])(p['enc_b1'])

    k2 = p['enc_k2']
    w2 = jnp.stack([k2[0, 0], k2[0, 1], k2[1, 0], k2[1, 1]])
    aff2 = _affine(p['enc_bn_g2'], p['enc_bn_b2'], p['enc_bn_m2'],
                   p['enc_bn_v2'])(p['enc_b2'])

    kd0 = p['dec_k0']
    wd0 = jnp.stack([kd0[0, 0], kd0[0, 1], kd0[1, 0], kd0[1, 1]])
    affd0 = _affine(p['dec_bn_g0'], p['dec_bn_b0'], p['dec_bn_m0'],
                    p['dec_bn_v0'])(p['dec_b0'])

    kd1 = p['dec_k1']
    wd1 = jnp.stack([
        kd1[0, 0], kd1[0, 1], kd1[1, 0], kd1[1, 1],
        kd1[0, 0] + kd1[0, 1], kd1[1, 0] + kd1[1, 1],
        kd1[0, 0] + kd1[1, 0], kd1[0, 1] + kd1[1, 1],
        kd1[0, 0] + kd1[0, 1] + kd1[1, 0] + kd1[1, 1],
    ])
    affd1 = _affine(p['dec_bn_g1'], p['dec_bn_b1'], p['dec_bn_m1'],
                    p['dec_bn_v1'])(p['dec_b1'])

    kf = p['out_k']
    rv_of = {(0, 0): 0, (0, 1): 1, (1, 0): 1, (1, 1): 1,
             (2, 0): 1, (2, 1): 2, (3, 0): 2, (3, 1): 2}
    wf = jnp.zeros((9, 96, 48), F32)
    for pp in range(4):
        for ss in range(4):
            for df in (0, 1):
                for dg in (0, 1):
                    rv = rv_of[(pp, df)]
                    cv = rv_of[(ss, dg)]
                    col = (pp * 4 + ss) * 3
                    wf = wf.at[rv * 3 + cv, :, col:col + 3].add(kf[df, dg])
    fb = jnp.tile(p['out_b'], 16).reshape(1, 48)

    xp = jnp.pad(x, ((0, 0), (0, 4), (0, 4), (0, 0)))
    p0 = (xp.reshape(N_IMG, 57, 4, 57, 4, 3)
            .transpose(0, 1, 3, 2, 4, 5)
            .reshape(N_IMG, 57, 57, 48))
    p1 = _conv_stage(_l0_kernel, p0, w0, aff0, (N_IMG, 57, 57, 512),
                     (N_IMG,), (1, 57, 57, 48),
                     pl.BlockSpec((1, 57, 57, 512), lambda n: (n, 0, 0, 0)))

    a1p = _conv_stage(_l1_kernel, p1, w1, aff1, (N_IMG, 57, 57, 96),
                      (N_IMG,), (1, 57, 57, 512),
                      pl.BlockSpec((1, 57, 57, 96), lambda n: (n, 0, 0, 0)))

    h = _conv_stage(_conv4_flat_kernel, a1p, w2, aff2, (N_IMG, 56, 5376),
                    (N_IMG,), (1, 57, 57, 96),
                    pl.BlockSpec((1, 56, 5376), lambda n: (n, 0, 0)))

    w3 = p['enc_dw'].reshape(56, 5376, LD)
    zq = pl.pallas_call(
        _dense_vq_kernel,
        grid=(7,),
        in_specs=[
            pl.BlockSpec((N_IMG, 8, 5376), lambda k: (0, k, 0)),
            pl.BlockSpec((8, 5376, LD), lambda k: (k, 0, 0)),
            pl.BlockSpec((1, LD), lambda k: (0, 0)),
            pl.BlockSpec((LD, NE), lambda k: (0, 0)),
        ],
        out_specs=pl.BlockSpec((N_IMG, LD), lambda k: (0, 0)),
        out_shape=jax.ShapeDtypeStruct((N_IMG, LD), F32),
    )(h, w3, p['enc_db'].reshape(1, LD), p['emb'])

    NB = FLAT // 12
    db3 = p['dec_db'].reshape(12, 1, NB)
    y = pl.pallas_call(
        _dense_dec_kernel,
        grid=(12,),
        in_specs=[
            pl.BlockSpec((N_IMG, LD), lambda k: (0, 0)),
            pl.BlockSpec((LD, NB), lambda k: (0, k)),
            pl.BlockSpec((1, 1, NB), lambda k: (k, 0, 0)),
        ],
        out_specs=pl.BlockSpec((N_IMG, NB), lambda k: (0, k)),
        out_shape=jax.ShapeDtypeStruct((N_IMG, FLAT), F32),
    )(zq, p['dec_dw'], db3)

    d0in = y.reshape(N_IMG, 56, 56, 96)
    d0 = _conv_stage(_conv4_padlow_kernel, d0in, wd0, affd0,
                     (N_IMG, 57, 57, 96),
                     (N_IMG,), (1, 56, 56, 96),
                     pl.BlockSpec((1, 57, 57, 96), lambda n: (n, 0, 0, 0)))

    d1 = _conv_stage(_dl1_kernel, d0, wd1, affd1, (N_IMG, 4, 57, 57, 96),
                     (N_IMG,), (1, 57, 57, 96),
                     pl.BlockSpec((1, 4, 57, 57, 96),
                                  lambda n: (n, 0, 0, 0, 0)))

    fin = d1
    out16 = pl.pallas_call(
        _final_kernel,
        grid=(N_IMG, 2),
        in_specs=[
            pl.BlockSpec((1, 4, 57, 57, 96), lambda n, s: (n, 0, 0, 0, 0)),
            pl.BlockSpec((9, 96, 48), lambda n, s: (0, 0, 0)),
            pl.BlockSpec((1, 48), lambda n, s: (0, 0)),
        ],
        out_specs=pl.BlockSpec((1, 28, 56, 48), lambda n, s: (n, s, 0, 0)),
        out_shape=jax.ShapeDtypeStruct((N_IMG, 56, 56, 48), F32),
    )(fin, wf, fb)

    out = (out16.reshape(N_IMG, 56, 56, 4, 4, 3)
                .transpose(0, 1, 3, 2, 4, 5)
                .reshape(N_IMG, 224, 224, 3))
    return out


# speedup vs baseline: 2.2889x; 2.2889x over previous
import jax
import jax.numpy as jnp
from jax import lax
from jax.experimental import pallas as pl

F32 = jnp.float32
HP = lax.Precision.HIGHEST
LP = lax.Precision.DEFAULT
N_IMG = 4
NF = 96
LD = 64
NE = 64
FLAT = 56 * 56 * 96


def _dot(a, b, prec=HP):
    return lax.dot_general(a, b, (((1,), (0,)), ((), ())), precision=prec,
                           preferred_element_type=F32)


def _l0_kernel(p_ref, w_ref, aff_ref, o_ref):
    ops = []
    for du in (0, 1):
        for dv in (0, 1):
            ops.append(p_ref[0, pl.ds(du, 56), pl.ds(dv, 56), :])
    g = jnp.concatenate(ops, axis=-1).reshape(56 * 56, 192)
    m = None
    for ab in range(4):
        y = _dot(g, w_ref[ab], LP)
        m = y if m is None else jnp.maximum(m, y)
    bias = aff_ref[0:1, :]
    scale = aff_ref[1:2, :]
    shift = aff_ref[2:3, :]
    res = jnp.maximum(m + bias, 0.0) * scale + shift
    o_ref[0, :56, :56, :] = res.reshape(56, 56, 512)
    o_ref[0, 56:57, :, :] = jnp.zeros((1, 57, 512), F32)
    o_ref[0, :, 56:57, :] = jnp.zeros((57, 1, 512), F32)


def _l1_kernel(p_ref, w_ref, aff_ref, o_ref):
    quads = []
    for a in (0, 1):
        for b in (0, 1):
            acc = None
            for di in (0, 1):
                for dj in (0, 1):
                    si, tj = a + di, b + dj
                    lane = ((si % 2) * 2 + (tj % 2)) * 128
                    op = p_ref[0, pl.ds(si // 2, 56),
                               pl.ds(tj // 2, 56), pl.ds(lane, 128)]
                    part = _dot(op.reshape(56 * 56, 128), w_ref[di * 2 + dj], LP)
                    acc = part if acc is None else acc + part
            quads.append(acc)
    m = jnp.maximum(jnp.maximum(quads[0], quads[1]),
                    jnp.maximum(quads[2], quads[3]))
    bias = aff_ref[0:1, :]
    scale = aff_ref[1:2, :]
    shift = aff_ref[2:3, :]
    res = jnp.maximum(m + bias, 0.0) * scale + shift
    o_ref[0, :56, :56, :] = res.reshape(56, 56, 96)
    o_ref[0, 56:57, :, :] = jnp.zeros((1, 57, 96), F32)
    o_ref[0, :, 56:57, :] = jnp.zeros((57, 1, 96), F32)


def _conv4_body(p_ref, w_ref, aff_ref, prec):
    acc = None
    for di in (0, 1):
        for dj in (0, 1):
            op = p_ref[0, pl.ds(di, 56), pl.ds(dj, 56), :]
            part = _dot(op.reshape(56 * 56, 96), w_ref[di * 2 + dj], prec)
            acc = part if acc is None else acc + part
    bias = aff_ref[0:1, :]
    scale = aff_ref[1:2, :]
    shift = aff_ref[2:3, :]
    return (jnp.maximum(acc + bias, 0.0) * scale + shift).reshape(56, 56, 96)


def _conv4_flat_kernel(p_ref, w_ref, aff_ref, o_ref):
    res = _conv4_body(p_ref, w_ref, aff_ref, LP)
    o_ref[0] = jnp.concatenate([res[:, j, :] for j in range(56)], axis=-1)


def _conv4_padlow_kernel(p_ref, w_ref, aff_ref, o_ref):
    x = p_ref[0]
    xp = jnp.concatenate([jnp.zeros((1, 56, 96), F32), x], axis=0)
    xp = jnp.concatenate([jnp.zeros((57, 1, 96), F32), xp], axis=1)
    acc = None
    for di in (0, 1):
        for dj in (0, 1):
            op = xp[di:di + 56, dj:dj + 56, :]
            part = _dot(op.reshape(56 * 56, 96), w_ref[di * 2 + dj], LP)
            acc = part if acc is None else acc + part
    bias = aff_ref[0:1, :]
    scale = aff_ref[1:2, :]
    shift = aff_ref[2:3, :]
    res = (jnp.maximum(acc + bias, 0.0) * scale + shift).reshape(56, 56, 96)
    o_ref[0, 1:57, 1:57, :] = res
    o_ref[0, 0:1, :, :] = jnp.zeros((1, 57, 96), F32)
    o_ref[0, :, 0:1, :] = jnp.zeros((57, 1, 96), F32)


def _dense_vq_kernel(h_ref, w_ref, db_ref, emb_ref, o_ref):
    k = pl.program_id(0)
    part = None
    for r in range(8):
        t = _dot(h_ref[:, r, :], w_ref[r], LP)
        part = t if part is None else part + t

    @pl.when(k == 0)
    def _():
        o_ref[...] = part

    @pl.when(k > 0)
    def _():
        o_ref[...] = o_ref[...] + part

    @pl.when(k == pl.num_programs(0) - 1)
    def _():
        z = o_ref[...] + db_ref[...]
        emb = emb_ref[...]
        e2 = jnp.sum(emb * emb, axis=0, keepdims=True)
        d = e2 - 2.0 * _dot(z, emb)
        mn = jnp.min(d, axis=1, keepdims=True)
        iota = lax.broadcasted_iota(jnp.int32, d.shape, 1)
        idx = jnp.min(jnp.where(d <= mn, iota, NE), axis=1, keepdims=True)
        onehot = (iota == idx).astype(F32)
        zq = lax.dot_general(onehot, emb, (((1,), (1,)), ((), ())),
                             precision=HP, preferred_element_type=F32)
        o_ref[...] = zq


def _dense_dec_kernel(z_ref, w_ref, db_ref, o_ref):
    y = _dot(z_ref[...], w_ref[...], LP) + db_ref[0]
    o_ref[...] = jnp.maximum(y, 0.0)


_DL1_TERMS = {
    (0, 0): [(0, 0, 0), (0, 1, 1), (1, 0, 2), (1, 1, 3)],
    (0, 1): [(0, 1, 4), (1, 1, 5)],
    (1, 0): [(1, 0, 6), (1, 1, 7)],
    (1, 1): [(1, 1, 8)],
}


def _dl1_kernel(p_ref, w_ref, aff_ref, o_ref):
    bias = aff_ref[0:1, :]
    scale = aff_ref[1:2, :]
    shift = aff_ref[2:3, :]
    for a in (0, 1):
        for b in (0, 1):
            acc = None
            for (ro, co, wi) in _DL1_TERMS[(a, b)]:
                op = p_ref[0, pl.ds(ro, 56), pl.ds(co, 56), :]
                part = _dot(op.reshape(56 * 56, 96), w_ref[wi], LP)
                acc = part if acc is None else acc + part
            res = jnp.maximum(acc + bias, 0.0) * scale + shift
            qi = a * 2 + b
            o_ref[0, qi, 1:57, 1:57, :] = res.reshape(56, 56, 96)
            o_ref[0, qi, 0:1, :, :] = jnp.zeros((1, 57, 96), F32)
            o_ref[0, qi, :, 0:1, :] = jnp.zeros((57, 1, 96), F32)


_ROWVARS = [(1, 0), (0, 1), (1, 1)]


def _final_kernel(p_ref, w_ref, fb_ref, o_ref):
    s = pl.program_id(1)
    rs = s * 28
    acc = None
    for rv, (a, ro) in enumerate(_ROWVARS):
        for cv, (b, co) in enumerate(_ROWVARS):
            op = p_ref[0, a * 2 + b, pl.ds(rs + ro, 28), pl.ds(co, 56), :]
            part = _dot(op.reshape(28 * 56, 96), w_ref[rv * 3 + cv], LP)
            acc = part if acc is None else acc + part
    y = acc + fb_ref[...]
    o_ref[0] = (1.0 / (1.0 + jnp.exp(-y))).reshape(28, 56, 48)


def _s2d(x):
    n, h, w, c = x.shape
    return (x.reshape(n, h // 2, 2, w // 2, 2, c)
             .transpose(0, 1, 3, 2, 4, 5)
             .reshape(n, h // 2, w // 2, 4 * c))


def _affine(g, b, m, v, pad=None):
    scale = g / jnp.sqrt(v + 1e-3)
    shift = b - m * scale
    if pad:
        z = jnp.zeros((pad - g.shape[0],), F32)
        return lambda bias: jnp.stack([jnp.concatenate([bias, z]),
                                       jnp.concatenate([scale, z]),
                                       jnp.concatenate([shift, z])])
    return lambda bias: jnp.stack([bias, scale, shift])


def _conv_stage(kfn, inp, w, aff, out_shape, grid, in_spec_shape, out_spec):
    ng = len(grid)

    def in_map(n, *r):
        return (n,) + (0,) * (len(in_spec_shape) - 1)

    return pl.pallas_call(
        kfn,
        grid=grid,
        in_specs=[
            pl.BlockSpec(in_spec_shape, in_map),
            pl.BlockSpec(w.shape, lambda *a: (0,) * w.ndim),
            pl.BlockSpec((3, aff.shape[1]), lambda *a: (0, 0)),
        ],
        out_specs=out_spec,
        out_shape=jax.ShapeDtypeStruct(out_shape, F32),
    )(inp, w, aff)


def kernel(x, params):
    p = params
    k0 = p['enc_k0']
    w0 = jnp.zeros((4, 192, 512), F32)
    for a in (0, 1):
        for b in (0, 1):
            for po_i in (0, 1):
                for po_j in (0, 1):
                    po = po_i * 2 + po_j
                    for di in (0, 1):
                        for dj in (0, 1):
                            s = 2 * po_i + a + di
                            t = 2 * po_j + b + dj
                            lane = ((s // 4) * 2 + (t // 4)) * 48 \
                                + ((s % 4) * 4 + (t % 4)) * 3
                            w0 = w0.at[a * 2 + b, lane:lane + 3,
                                       po * 128:po * 128 + 96].add(k0[di, dj])
    aff0 = _affine(p['enc_bn_g0'], p['enc_bn_b0'], p['enc_bn_m0'],
                   p['enc_bn_v0'], pad=128)(p['enc_b0'])
    aff0 = jnp.tile(aff0, (1, 4))

    k1 = p['enc_k1']
    w1 = jnp.zeros((4, 128, 96), F32)
    for di in (0, 1):
        for dj in (0, 1):
            w1 = w1.at[di * 2 + dj, :96, :].set(k1[di, dj])
    aff1 = _affine(p['enc_bn_g1'], p['enc_bn_b1'], p['enc_bn_m1'],
                   p['enc_bn_v1'])(p['enc_b1'])

    k2 = p['enc_k2']
    w2 = jnp.stack([k2[0, 0], k2[0, 1], k2[1, 0], k2[1, 1]])
    aff2 = _affine(p['enc_bn_g2'], p['enc_bn_b2'], p['enc_bn_m2'],
                   p['enc_bn_v2'])(p['enc_b2'])

    kd0 = p['dec_k0']
    wd0 = jnp.stack([kd0[0, 0], kd0[0, 1], kd0[1, 0], kd0[1, 1]])
    affd0 = _affine(p['dec_bn_g0'], p['dec_bn_b0'], p['dec_bn_m0'],
                    p['dec_bn_v0'])(p['dec_b0'])

    kd1 = p['dec_k1']
    wd1 = jnp.stack([
        kd1[0, 0], kd1[0, 1], kd1[1, 0], kd1[1, 1],
        kd1[0, 0] + kd1[0, 1], kd1[1, 0] + kd1[1, 1],
        kd1[0, 0] + kd1[1, 0], kd1[0, 1] + kd1[1, 1],
        kd1[0, 0] + kd1[0, 1] + kd1[1, 0] + kd1[1, 1],
    ])
    affd1 = _affine(p['dec_bn_g1'], p['dec_bn_b1'], p['dec_bn_m1'],
                    p['dec_bn_v1'])(p['dec_b1'])

    kf = p['out_k']
    rv_of = {(0, 0): 0, (0, 1): 1, (1, 0): 1, (1, 1): 1,
             (2, 0): 1, (2, 1): 2, (3, 0): 2, (3, 1): 2}
    wf = jnp.zeros((9, 96, 48), F32)
    for pp in range(4):
        for ss in range(4):
            for df in (0, 1):
                for dg in (0, 1):
                    rv = rv_of[(pp, df)]
                    cv = rv_of[(ss, dg)]
                    col = (pp * 4 + ss) * 3
                    wf = wf.at[rv * 3 + cv, :, col:col + 3].add(kf[df, dg])
    fb = jnp.tile(p['out_b'], 16).reshape(1, 48)

    xp = jnp.pad(x, ((0, 0), (0, 4), (0, 4), (0, 0)))
    p0 = (xp.reshape(N_IMG, 57, 4, 57, 4, 3)
            .transpose(0, 1, 3, 2, 4, 5)
            .reshape(N_IMG, 57, 57, 48))
    p1 = _conv_stage(_l0_kernel, p0, w0, aff0, (N_IMG, 57, 57, 512),
                     (N_IMG,), (1, 57, 57, 48),
                     pl.BlockSpec((1, 57, 57, 512), lambda n: (n, 0, 0, 0)))

    a1p = _conv_stage(_l1_kernel, p1, w1, aff1, (N_IMG, 57, 57, 96),
                      (N_IMG,), (1, 57, 57, 512),
                      pl.BlockSpec((1, 57, 57, 96), lambda n: (n, 0, 0, 0)))

    h = _conv_stage(_conv4_flat_kernel, a1p, w2, aff2, (N_IMG, 56, 5376),
                    (N_IMG,), (1, 57, 57, 96),
                    pl.BlockSpec((1, 56, 5376), lambda n: (n, 0, 0)))

    w3 = p['enc_dw'].reshape(56, 5376, LD)
    zq = pl.pallas_call(
        _dense_vq_kernel,
        grid=(7,),
        in_specs=[
            pl.BlockSpec((N_IMG, 8, 5376), lambda k: (0, k, 0)),
            pl.BlockSpec((8, 5376, LD), lambda k: (k, 0, 0)),
            pl.BlockSpec((1, LD), lambda k: (0, 0)),
            pl.BlockSpec((LD, NE), lambda k: (0, 0)),
        ],
        out_specs=pl.BlockSpec((N_IMG, LD), lambda k: (0, 0)),
        out_shape=jax.ShapeDtypeStruct((N_IMG, LD), F32),
    )(h, w3, p['enc_db'].reshape(1, LD), p['emb'])

    NB = FLAT // 12
    db3 = p['dec_db'].reshape(12, 1, NB)
    y = pl.pallas_call(
        _dense_dec_kernel,
        grid=(12,),
        in_specs=[
            pl.BlockSpec((N_IMG, LD), lambda k: (0, 0)),
            pl.BlockSpec((LD, NB), lambda k: (0, k)),
            pl.BlockSpec((1, 1, NB), lambda k: (k, 0, 0)),
        ],
        out_specs=pl.BlockSpec((N_IMG, NB), lambda k: (0, k)),
        out_shape=jax.ShapeDtypeStruct((N_IMG, FLAT), F32),
    )(zq, p['dec_dw'], db3)

    d0in = y.reshape(N_IMG, 56, 56, 96)
    d0 = _conv_stage(_conv4_padlow_kernel, d0in, wd0, affd0,
                     (N_IMG, 57, 57, 96),
                     (N_IMG,), (1, 56, 56, 96),
                     pl.BlockSpec((1, 57, 57, 96), lambda n: (n, 0, 0, 0)))

    d1 = _conv_stage(_dl1_kernel, d0, wd1, affd1, (N_IMG, 4, 57, 57, 96),
                     (N_IMG,), (1, 57, 57, 96),
                     pl.BlockSpec((1, 4, 57, 57, 96),
                                  lambda n: (n, 0, 0, 0, 0)))

    fin = d1
    out16 = pl.pallas_call(
        _final_kernel,
        grid=(N_IMG, 2),
        in_specs=[
            pl.BlockSpec((1, 4, 57, 57, 96), lambda n, s: (n, 0, 0, 0, 0)),
            pl.BlockSpec((9, 96, 48), lambda n, s: (0, 0, 0)),
            pl.BlockSpec((1, 48), lambda n, s: (0, 0)),
        ],
        out_specs=pl.BlockSpec((1, 28, 56, 48), lambda n, s: (n, s, 0, 0)),
        out_shape=jax.ShapeDtypeStruct((N_IMG, 56, 56, 48), F32),
    )(fin, wf, fb)

    out = (out16.reshape(N_IMG, 56, 56, 4, 4, 3)
                .transpose(0, 1, 3, 2, 4, 5)
                .reshape(N_IMG, 224, 224, 3))
    return out
